# baseline (speedup 1.0000x reference)
import functools

import jax
import jax.numpy as jnp
import numpy as np
from jax.experimental import pallas as pl
from jax.experimental.pallas import tpu as pltpu
from jax.experimental.pallas import tpu_sc as plsc

EPS = float(np.finfo(np.float32).eps)
K = 8192
N = 256

_VECTOR_MESH = plsc.VectorSubcoreMesh(core_axis_name="core", subcore_axis_name="subcore")
_GATHER_WIN = 128


def _sc_gather(emb, idx):
    n_tok = idx.shape[0]
    idx2 = idx.reshape(1, n_tok)

    @pl.kernel(out_type=jax.ShapeDtypeStruct((n_tok, N), emb.dtype), mesh=_VECTOR_MESH)
    def kern(emb_hbm, i_hbm, o_hbm):
        def body(i_vmem, o_vmem):
            pltpu.sync_copy(emb_hbm.at[i_vmem.at[0]], o_vmem)

        pltpu.emit_pipeline(
            body,
            grid=(n_tok // _GATHER_WIN,),
            in_specs=[pl.BlockSpec((1, _GATHER_WIN), index_map=lambda i: (0, i))],
            out_specs=[pl.BlockSpec((_GATHER_WIN, N), index_map=lambda i: (i, 0))],
            core_axis_name=("core", "subcore"),
            dimension_semantics=(pltpu.PARALLEL,),
        )(i_hbm, o_hbm)

    return kern(emb, idx2)


def _layer_kernel(nblk, bt, q_ref, idx_ref, prev_ref,
                  cum_ref, counts_ref, ppl_ref):
    i = pl.program_id(0)
    idxv = idx_ref[0, 0, :]
    qt = jnp.transpose(q_ref[...], (1, 0))
    cum_ref[...] = prev_ref[...] + qt[None]

    @pl.when(i == 0)
    def _init():
        counts_ref[...] = jnp.zeros_like(counts_ref)
        ppl_ref[...] = jnp.zeros_like(ppl_ref)

    kc = 1024
    tc = 512
    for c in range(K // kc):
        ids = jax.lax.broadcasted_iota(jnp.int32, (tc, kc), 1) + c * kc
        acc = jnp.zeros((1, kc), jnp.float32)
        for t0 in range(0, bt, tc):
            oh = (idxv[t0:t0 + tc, None] == ids).astype(jnp.float32)
            acc = acc + jnp.sum(oh, axis=0, keepdims=True)
        counts_ref[0:1, c * kc:(c + 1) * kc] += acc

    @pl.when(i == nblk - 1)
    def _ppl():
        p = counts_ref[...] * (1.0 / (nblk * bt))
        ppl_ref[...] = jnp.exp(-jnp.sum(p * jnp.log(p + EPS))).reshape(1, 1)


def _cum_stats_layer(q, idx, prev_cum):
    B, Nd, T = prev_cum.shape
    bt = T
    nblk = B
    idx3 = idx.reshape(nblk, 1, bt)
    kern = functools.partial(_layer_kernel, nblk, bt)
    cum, counts, ppl = pl.pallas_call(
        kern,
        grid=(nblk,),
        in_specs=[
            pl.BlockSpec((bt, Nd), lambda i: (i, 0)),
            pl.BlockSpec((1, 1, bt), lambda i: (i, 0, 0)),
            pl.BlockSpec((1, Nd, bt), lambda i: (i, 0, 0)),
        ],
        out_specs=[
            pl.BlockSpec((1, Nd, bt), lambda i: (i, 0, 0)),
            pl.BlockSpec((1, K), lambda i: (0, 0)),
            pl.BlockSpec((1, 1), lambda i: (0, 0)),
        ],
        out_shape=[
            jax.ShapeDtypeStruct((B, Nd, T), jnp.float32),
            jax.ShapeDtypeStruct((1, K), jnp.float32),
            jax.ShapeDtypeStruct((1, 1), jnp.float32),
        ],
    )(q, idx3, prev_cum)
    return cum, ppl[0, 0]


def kernel(input, emb0, emb1, emb2):
    embs = [emb0, emb1, emb2]
    B, Nd, T = input.shape
    residual = input
    indices_list = []
    for i, e in enumerate(embs):
        x_detach = jax.lax.stop_gradient(residual)
        flat = jnp.transpose(x_detach, (0, 2, 1)).reshape(B * T, Nd)
        eu_dis = (jnp.sum(flat ** 2, axis=-1, keepdims=True)
                  + jnp.sum(e ** 2, axis=-1)[None, :]
                  - 2.0 * flat @ e.T)
        idx = jnp.argmin(eu_dis, axis=-1)
        quantized = jnp.take(e, idx, axis=0)
        quantized = jnp.transpose(quantized.reshape(B, T, Nd), (0, 2, 1))
        residual = residual - quantized
        indices_list.append(idx)

    cum = jnp.zeros_like(input)
    cums = []
    ppl_list = []
    for i, e in enumerate(embs):
        q = _sc_gather(e, indices_list[i])
        cum, ppl = _cum_stats_layer(q, indices_list[i], cum)
        cums.append(cum)
        ppl_list.append(ppl)
    quantized = jnp.stack(cums, axis=-1)
    indices = jnp.stack([ix.reshape(B, T) for ix in indices_list], axis=-1)
    ppl = jnp.stack(ppl_list, axis=-1)
    return quantized, indices, ppl


# speedup vs baseline: 1.1643x; 1.1643x over previous
import functools

import jax
import jax.numpy as jnp
import numpy as np
from jax.experimental import pallas as pl
from jax.experimental.pallas import tpu as pltpu
from jax.experimental.pallas import tpu_sc as plsc

EPS = float(np.finfo(np.float32).eps)
K = 8192
N = 256
HI = 64
LO = 128

_VECTOR_MESH = plsc.VectorSubcoreMesh(core_axis_name="core", subcore_axis_name="subcore")
_GATHER_WIN = 128


def _sc_gather(emb, idx):
    n_tok = idx.shape[0]
    idx2 = idx.reshape(1, n_tok)

    @pl.kernel(out_type=jax.ShapeDtypeStruct((n_tok, N), emb.dtype), mesh=_VECTOR_MESH)
    def kern(emb_hbm, i_hbm, o_hbm):
        def body(i_vmem, o_vmem):
            pltpu.sync_copy(emb_hbm.at[i_vmem.at[0]], o_vmem)

        pltpu.emit_pipeline(
            body,
            grid=(n_tok // _GATHER_WIN,),
            in_specs=[pl.BlockSpec((1, _GATHER_WIN), index_map=lambda i: (0, i))],
            out_specs=[pl.BlockSpec((_GATHER_WIN, N), index_map=lambda i: (i, 0))],
            core_axis_name=("core", "subcore"),
            dimension_semantics=(pltpu.PARALLEL,),
        )(i_hbm, o_hbm)

    return kern(emb, idx2)


def _stats_kernel(nblk, bt, q0_ref, q1_ref, q2_ref, i0_ref, i1_ref, i2_ref,
                  c0_ref, c1_ref, c2_ref, counts_ref, ppl_ref):
    i = pl.program_id(0)
    c0 = jnp.transpose(q0_ref[...], (1, 0))
    c1 = c0 + jnp.transpose(q1_ref[...], (1, 0))
    c2 = c1 + jnp.transpose(q2_ref[...], (1, 0))
    c0_ref[...] = c0[None]
    c1_ref[...] = c1[None]
    c2_ref[...] = c2[None]

    @pl.when(i == 0)
    def _init():
        counts_ref[...] = jnp.zeros_like(counts_ref)
        ppl_ref[...] = jnp.zeros_like(ppl_ref)

    iota_hi = jax.lax.broadcasted_iota(jnp.int32, (bt, HI), 1)
    iota_lo = jax.lax.broadcasted_iota(jnp.int32, (bt, LO), 1)
    for l, idx_ref in enumerate((i0_ref, i1_ref, i2_ref)):
        idxv = idx_ref[0, 0, :]
        oh_hi = (jax.lax.shift_right_logical(idxv, 7)[:, None] == iota_hi).astype(jnp.float32)
        oh_lo = ((idxv & (LO - 1))[:, None] == iota_lo).astype(jnp.float32)
        c2d = jax.lax.dot_general(oh_hi, oh_lo, (((0,), (0,)), ((), ())),
                                  preferred_element_type=jnp.float32)
        counts_ref[l] += c2d

    @pl.when(i == nblk - 1)
    def _ppl():
        ppls = []
        for l in range(3):
            p = counts_ref[l] * (1.0 / (nblk * bt))
            ppls.append(jnp.exp(-jnp.sum(p * jnp.log(p + EPS))))
        ppl_ref[...] = jnp.stack(ppls).reshape(1, 3)


def _cum_stats(q0, q1, q2, i0, i1, i2, B, Nd, T):
    bt = T
    nblk = B
    idx_specs = pl.BlockSpec((1, 1, bt), lambda i: (i, 0, 0))
    q_spec = pl.BlockSpec((bt, Nd), lambda i: (i, 0))
    cum_spec = pl.BlockSpec((1, Nd, bt), lambda i: (i, 0, 0))
    kern = functools.partial(_stats_kernel, nblk, bt)
    c0, c1, c2, counts, ppl = pl.pallas_call(
        kern,
        grid=(nblk,),
        in_specs=[q_spec, q_spec, q_spec, idx_specs, idx_specs, idx_specs],
        out_specs=[
            cum_spec, cum_spec, cum_spec,
            pl.BlockSpec((3, HI, LO), lambda i: (0, 0, 0)),
            pl.BlockSpec((1, 3), lambda i: (0, 0)),
        ],
        out_shape=[
            jax.ShapeDtypeStruct((B, Nd, T), jnp.float32),
            jax.ShapeDtypeStruct((B, Nd, T), jnp.float32),
            jax.ShapeDtypeStruct((B, Nd, T), jnp.float32),
            jax.ShapeDtypeStruct((3, HI, LO), jnp.float32),
            jax.ShapeDtypeStruct((1, 3), jnp.float32),
        ],
    )(q0, q1, q2,
      i0.reshape(nblk, 1, bt), i1.reshape(nblk, 1, bt), i2.reshape(nblk, 1, bt))
    return (c0, c1, c2), ppl[0]


def kernel(input, emb0, emb1, emb2):
    embs = [emb0, emb1, emb2]
    B, Nd, T = input.shape
    residual = input
    indices_list = []
    for i, e in enumerate(embs):
        x_detach = jax.lax.stop_gradient(residual)
        flat = jnp.transpose(x_detach, (0, 2, 1)).reshape(B * T, Nd)
        eu_dis = (jnp.sum(flat ** 2, axis=-1, keepdims=True)
                  + jnp.sum(e ** 2, axis=-1)[None, :]
                  - 2.0 * flat @ e.T)
        idx = jnp.argmin(eu_dis, axis=-1)
        quantized = jnp.take(e, idx, axis=0)
        quantized = jnp.transpose(quantized.reshape(B, T, Nd), (0, 2, 1))
        residual = residual - quantized
        indices_list.append(idx)

    qs = [_sc_gather(e, ix) for e, ix in zip(embs, indices_list)]
    cums, ppl = _cum_stats(qs[0], qs[1], qs[2],
                           indices_list[0], indices_list[1], indices_list[2],
                           B, Nd, T)
    quantized = jnp.stack(cums, axis=-1)
    indices = jnp.stack([ix.reshape(B, T) for ix in indices_list], axis=-1)
    return quantized, indices, ppl


# speedup vs baseline: 1.1954x; 1.0266x over previous
import functools

import jax
import jax.numpy as jnp
import numpy as np
from jax.experimental import pallas as pl
from jax.experimental.pallas import tpu as pltpu
from jax.experimental.pallas import tpu_sc as plsc

EPS = float(np.finfo(np.float32).eps)
K = 8192
N = 256
HI = 64
LO = 128

_VECTOR_MESH = plsc.VectorSubcoreMesh(core_axis_name="core", subcore_axis_name="subcore")
_GATHER_WIN = 128


def _sc_gather(emb, idx):
    n_tok = idx.shape[0]
    idx2 = idx.reshape(1, n_tok)

    @pl.kernel(out_type=jax.ShapeDtypeStruct((n_tok, N), emb.dtype), mesh=_VECTOR_MESH)
    def kern(emb_hbm, i_hbm, o_hbm):
        def body(i_vmem, o_vmem):
            pltpu.sync_copy(emb_hbm.at[i_vmem.at[0]], o_vmem)

        pltpu.emit_pipeline(
            body,
            grid=(n_tok // _GATHER_WIN,),
            in_specs=[pl.BlockSpec((1, _GATHER_WIN), index_map=lambda i: (0, i))],
            out_specs=[pl.BlockSpec((_GATHER_WIN, N), index_map=lambda i: (i, 0))],
            core_axis_name=("core", "subcore"),
            dimension_semantics=(pltpu.PARALLEL,),
        )(i_hbm, o_hbm)

    return kern(emb, idx2)


def _stats_kernel(nblk, bt, q0_ref, q1_ref, q2_ref, i0_ref, i1_ref, i2_ref,
                  c0_ref, c1_ref, c2_ref, counts_ref, ppl_ref):
    i = pl.program_id(0)
    c0 = jnp.transpose(q0_ref[...], (1, 0))
    c1 = c0 + jnp.transpose(q1_ref[...], (1, 0))
    c2 = c1 + jnp.transpose(q2_ref[...], (1, 0))
    c0_ref[...] = c0[None]
    c1_ref[...] = c1[None]
    c2_ref[...] = c2[None]

    @pl.when(i == 0)
    def _init():
        counts_ref[...] = jnp.zeros_like(counts_ref)
        ppl_ref[...] = jnp.zeros_like(ppl_ref)

    iota_hi = jax.lax.broadcasted_iota(jnp.int32, (bt, HI), 1)
    iota_lo = jax.lax.broadcasted_iota(jnp.int32, (bt, LO), 1)
    for l, idx_ref in enumerate((i0_ref, i1_ref, i2_ref)):
        idxv = idx_ref[0, 0, :]
        oh_hi = (jax.lax.shift_right_logical(idxv, 7)[:, None] == iota_hi).astype(jnp.float32)
        oh_lo = ((idxv & (LO - 1))[:, None] == iota_lo).astype(jnp.float32)
        c2d = jax.lax.dot_general(oh_hi, oh_lo, (((0,), (0,)), ((), ())),
                                  preferred_element_type=jnp.float32)
        counts_ref[l] += c2d

    @pl.when(i == nblk - 1)
    def _ppl():
        ppls = []
        for l in range(3):
            p = counts_ref[l] * (1.0 / (nblk * bt))
            ppls.append(jnp.exp(-jnp.sum(p * jnp.log(p + EPS))))
        ppl_ref[...] = jnp.stack(ppls).reshape(1, 3)


def _cum_stats(q0, q1, q2, i0, i1, i2, B, Nd, T):
    bt = T
    nblk = B
    idx_specs = pl.BlockSpec((1, 1, bt), lambda i: (i, 0, 0))
    q_spec = pl.BlockSpec((bt, Nd), lambda i: (i, 0))
    cum_spec = pl.BlockSpec((1, Nd, bt), lambda i: (i, 0, 0))
    kern = functools.partial(_stats_kernel, nblk, bt)
    c0, c1, c2, counts, ppl = pl.pallas_call(
        kern,
        grid=(nblk,),
        in_specs=[q_spec, q_spec, q_spec, idx_specs, idx_specs, idx_specs],
        out_specs=[
            cum_spec, cum_spec, cum_spec,
            pl.BlockSpec((3, HI, LO), lambda i: (0, 0, 0)),
            pl.BlockSpec((1, 3), lambda i: (0, 0)),
        ],
        out_shape=[
            jax.ShapeDtypeStruct((B, Nd, T), jnp.float32),
            jax.ShapeDtypeStruct((B, Nd, T), jnp.float32),
            jax.ShapeDtypeStruct((B, Nd, T), jnp.float32),
            jax.ShapeDtypeStruct((3, HI, LO), jnp.float32),
            jax.ShapeDtypeStruct((1, 3), jnp.float32),
        ],
    )(q0, q1, q2,
      i0.reshape(nblk, 1, bt), i1.reshape(nblk, 1, bt), i2.reshape(nblk, 1, bt))
    return (c0, c1, c2), ppl[0]


def kernel(input, emb0, emb1, emb2):
    embs = [emb0, emb1, emb2]
    B, Nd, T = input.shape
    residual = input
    indices_list = []
    qs = []
    for i, e in enumerate(embs):
        x_detach = jax.lax.stop_gradient(residual)
        flat = jnp.transpose(x_detach, (0, 2, 1)).reshape(B * T, Nd)
        eu_dis = (jnp.sum(flat ** 2, axis=-1, keepdims=True)
                  + jnp.sum(e ** 2, axis=-1)[None, :]
                  - 2.0 * flat @ e.T)
        idx = jnp.argmin(eu_dis, axis=-1)
        q = _sc_gather(e, idx)
        quantized = jnp.transpose(q.reshape(B, T, Nd), (0, 2, 1))
        residual = residual - quantized
        indices_list.append(idx)
        qs.append(q)
    cums, ppl = _cum_stats(qs[0], qs[1], qs[2],
                           indices_list[0], indices_list[1], indices_list[2],
                           B, Nd, T)
    quantized = jnp.stack(cums, axis=-1)
    indices = jnp.stack([ix.reshape(B, T) for ix in indices_list], axis=-1)
    return quantized, indices, ppl
